# Initial kernel scaffold; baseline (speedup 1.0000x reference)
import jax
import jax.numpy as jnp
from jax import lax
from jax.experimental import pallas as pl
from jax.experimental.pallas import tpu as pltpu
from jax.experimental.pallas import tpu_sc as plsc

_B = 16384
_F = 26
_V = 100000
_D = 32
_NUM = 13
_E = _F * _D

_NC = 2
_NS = 16
_NW = _NC * _NS
_TOTAL = _B * _F
_PER_W = _TOTAL // _NW
_CH = 1664
_NCH = _PER_W // _CH


def _gather_body(tab_ref, idx_ref, out_ref, idx_v, rows_v, sem):
    wid = lax.axis_index("s") * _NC + lax.axis_index("c")
    w_base = wid * _PER_W

    def chunk(j, carry):
        base = w_base + j * _CH
        pltpu.sync_copy(idx_ref.at[pl.ds(base, _CH)], idx_v)
        pltpu.async_copy(tab_ref.at[idx_v], rows_v, sem).wait()
        pltpu.sync_copy(rows_v, out_ref.at[pl.ds(base, _CH)])
        return carry

    lax.fori_loop(0, _NCH, chunk, 0)


_sc_gather = pl.kernel(
    _gather_body,
    out_type=jax.ShapeDtypeStruct((_TOTAL, _D), jnp.float32),
    mesh=plsc.VectorSubcoreMesh(core_axis_name="c", subcore_axis_name="s"),
    scratch_types=[
        pltpu.VMEM((_CH,), jnp.int32),
        pltpu.VMEM((_CH, _D), jnp.float32),
        pltpu.SemaphoreType.DMA,
    ],
)

_BB = 1024


def _mlp_body(emb_ref, num_ref, w1a_ref, w1b_ref, b1_ref, w2_ref, b2_ref, out_ref):
    h = jnp.dot(emb_ref[...], w1a_ref[...], preferred_element_type=jnp.float32)
    h = h + jnp.dot(num_ref[...], w1b_ref[...], preferred_element_type=jnp.float32)
    h = jnp.maximum(h + b1_ref[...], 0.0)
    o = jnp.dot(h, w2_ref[...], preferred_element_type=jnp.float32)
    out_ref[...] = jnp.maximum(o + b2_ref[...], 0.0)


def _mlp(emb, num, w1a, w1b, b1, w2, b2, interpret=False):
    return pl.pallas_call(
        _mlp_body,
        grid=(_B // _BB,),
        in_specs=[
            pl.BlockSpec((_BB, _E), lambda i: (i, 0)),
            pl.BlockSpec((_BB, _NUM), lambda i: (i, 0)),
            pl.BlockSpec((_E, 64), lambda i: (0, 0)),
            pl.BlockSpec((_NUM, 64), lambda i: (0, 0)),
            pl.BlockSpec((1, 64), lambda i: (0, 0)),
            pl.BlockSpec((64, 32), lambda i: (0, 0)),
            pl.BlockSpec((1, 32), lambda i: (0, 0)),
        ],
        out_specs=pl.BlockSpec((_BB, 32), lambda i: (i, 0)),
        out_shape=jax.ShapeDtypeStruct((_B, 32), jnp.float32),
        interpret=interpret,
    )(emb, num, w1a, w1b, b1, w2, b2)


def kernel(cate_inputs, num_inputs, tables, W1, b1, W2, b2):
    idx = cate_inputs.astype(jnp.int32) + (jnp.arange(_F, dtype=jnp.int32) * _V)[None, :]
    emb = _sc_gather(tables.reshape(_F * _V, _D), idx.reshape(-1))
    emb = emb.reshape(_B, _E)
    return _mlp(emb, num_inputs, W1[:_E], W1[_E:], b1.reshape(1, 64),
                W2, b2.reshape(1, 32))


# speedup vs baseline: 8.0518x; 8.0518x over previous
import jax
import jax.numpy as jnp
from jax import lax
from jax.experimental import pallas as pl
from jax.experimental.pallas import tpu as pltpu
from jax.experimental.pallas import tpu_sc as plsc

_B = 16384
_F = 26
_V = 100000
_D = 32
_NUM = 13
_E = _F * _D

_NC = 2
_NS = 16
_NW = _NC * _NS
_TOTAL = _B * _F
_PER_W = _TOTAL // _NW
_CH = 1664
_NCH = _PER_W // _CH


def _gather_body(tab_ref, idx_ref, out_ref, idx_v, rows_v, sem):
    wid = lax.axis_index("s") * _NC + lax.axis_index("c")
    w_base = wid * _PER_W

    def chunk(j, carry):
        base = w_base + j * _CH
        pltpu.sync_copy(idx_ref.at[pl.ds(base, _CH)], idx_v)
        pltpu.async_copy(tab_ref.at[idx_v], rows_v, sem).wait()
        pltpu.sync_copy(rows_v, out_ref.at[pl.ds(base, _CH)])
        return carry

    lax.fori_loop(0, _NCH, chunk, 0)


import functools


@functools.cache
def _sc_gather_fn():
    return pl.kernel(
        _gather_body,
        out_type=jax.ShapeDtypeStruct((_TOTAL, _D), jnp.float32),
        mesh=plsc.VectorSubcoreMesh(core_axis_name="c", subcore_axis_name="s"),
        scratch_types=[
            pltpu.VMEM((_CH,), jnp.int32),
            pltpu.VMEM((_CH, _D), jnp.float32),
            pltpu.SemaphoreType.DMA,
        ],
        compiler_params=pltpu.CompilerParams(use_tc_tiling_on_sc=False),
    )

_BB = 1024


def _mlp_body(emb_ref, num_ref, w1a_ref, w1b_ref, b1_ref, w2_ref, b2_ref, out_ref):
    h = jnp.dot(emb_ref[...], w1a_ref[...], preferred_element_type=jnp.float32)
    h = h + jnp.dot(num_ref[...], w1b_ref[...], preferred_element_type=jnp.float32)
    h = jnp.maximum(h + b1_ref[...], 0.0)
    o = jnp.dot(h, w2_ref[...], preferred_element_type=jnp.float32)
    out_ref[...] = jnp.maximum(o + b2_ref[...], 0.0)


def _mlp(emb, num, w1a, w1b, b1, w2, b2, interpret=False):
    return pl.pallas_call(
        _mlp_body,
        grid=(_B // _BB,),
        in_specs=[
            pl.BlockSpec((_BB, _E), lambda i: (i, 0)),
            pl.BlockSpec((_BB, _NUM), lambda i: (i, 0)),
            pl.BlockSpec((_E, 64), lambda i: (0, 0)),
            pl.BlockSpec((_NUM, 64), lambda i: (0, 0)),
            pl.BlockSpec((1, 64), lambda i: (0, 0)),
            pl.BlockSpec((64, 32), lambda i: (0, 0)),
            pl.BlockSpec((1, 32), lambda i: (0, 0)),
        ],
        out_specs=pl.BlockSpec((_BB, 32), lambda i: (i, 0)),
        out_shape=jax.ShapeDtypeStruct((_B, 32), jnp.float32),
        interpret=interpret,
    )(emb, num, w1a, w1b, b1, w2, b2)


def kernel(cate_inputs, num_inputs, tables, W1, b1, W2, b2):
    idx = cate_inputs.astype(jnp.int32) + (jnp.arange(_F, dtype=jnp.int32) * _V)[None, :]
    emb = _sc_gather_fn()(tables.reshape(_F * _V, _D), idx.reshape(-1))
    emb = emb.reshape(_B, _E)
    return _mlp(emb, num_inputs, W1[:_E], W1[_E:], b1.reshape(1, 64),
                W2, b2.reshape(1, 32))


# speedup vs baseline: 27.2800x; 3.3880x over previous
import jax
import jax.numpy as jnp
from jax import lax
from jax.experimental import pallas as pl
from jax.experimental.pallas import tpu as pltpu
from jax.experimental.pallas import tpu_sc as plsc

_B = 16384
_F = 26
_V = 100000
_D = 32
_NUM = 13
_E = _F * _D

_NC = 2
_NS = 16
_NW = _NC * _NS
_TOTAL = _B * _F
_PER_W = _TOTAL // _NW
_CH = 1664
_NCH = _PER_W // _CH


def _gather_body(tab_ref, idx_ref, out_ref, idx_v, rows_v, sem):
    wid = lax.axis_index("s") * _NC + lax.axis_index("c")
    w_base = wid * _PER_W

    def chunk(j, carry):
        base = w_base + j * _CH
        pltpu.sync_copy(idx_ref.at[pl.ds(base, _CH)], idx_v)
        pltpu.async_copy(tab_ref.at[idx_v], rows_v, sem).wait()
        pltpu.sync_copy(rows_v, out_ref.at[pl.ds(base, _CH)])
        return carry

    lax.fori_loop(0, _NCH, chunk, 0)


import functools


@functools.cache
def _sc_gather_fn():
    return pl.kernel(
        _gather_body,
        out_type=jax.ShapeDtypeStruct((_TOTAL, _D), jnp.float32),
        mesh=plsc.VectorSubcoreMesh(core_axis_name="c", subcore_axis_name="s"),
        scratch_types=[
            pltpu.VMEM((_CH,), jnp.int32),
            pltpu.VMEM((_CH, _D), jnp.float32),
            pltpu.SemaphoreType.DMA,
        ],
        compiler_params=pltpu.CompilerParams(use_tc_tiling_on_sc=False),
    )

_VB = 50048
_NVB = 2
_VP = _VB * _NVB
_VQ = _VB // 4


_DN = (((0,), (0,)), ((), ()))


def _fmt_body(src_ref, out_ref):
    x = src_ref[0]
    j = pl.program_id(1)
    col = jax.lax.broadcasted_iota(jnp.int32, (_D, _VB), 1) + j * _VB
    x = jnp.where(col < _V, x, 0.0)
    xs = jnp.concatenate([x[:, c * _VQ:(c + 1) * _VQ] for c in range(4)], axis=0)
    out_ref[...] = jax.lax.dot_general(xs, jnp.eye(128, dtype=jnp.float32), _DN,
                                       preferred_element_type=jnp.float32)


def _fmt(tt3, interpret=False):
    return pl.pallas_call(
        _fmt_body,
        grid=(_F, _NVB),
        in_specs=[pl.BlockSpec((1, _D, _VB), lambda f, j: (f, 0, j))],
        out_specs=pl.BlockSpec((_VB // 4, 128), lambda f, j: (f * _NVB + j, 0)),
        out_shape=jax.ShapeDtypeStruct((_F * _VP // 4, 128), jnp.float32),
        interpret=interpret,
    )(tt3)


_BB = 1024


def _mlp_body(emb_ref, num_ref, w1a_ref, w1b_ref, b1_ref, w2_ref, b2_ref, out_ref):
    h = jnp.dot(emb_ref[...], w1a_ref[...], preferred_element_type=jnp.float32)
    h = h + jnp.dot(num_ref[...], w1b_ref[...], preferred_element_type=jnp.float32)
    h = jnp.maximum(h + b1_ref[...], 0.0)
    o = jnp.dot(h, w2_ref[...], preferred_element_type=jnp.float32)
    out_ref[...] = jnp.maximum(o + b2_ref[...], 0.0)


def _mlp(emb, num, w1a, w1b, b1, w2, b2, interpret=False):
    return pl.pallas_call(
        _mlp_body,
        grid=(_B // _BB,),
        in_specs=[
            pl.BlockSpec((_BB, _E), lambda i: (i, 0)),
            pl.BlockSpec((_BB, _NUM), lambda i: (i, 0)),
            pl.BlockSpec((_E, 64), lambda i: (0, 0)),
            pl.BlockSpec((_NUM, 64), lambda i: (0, 0)),
            pl.BlockSpec((1, 64), lambda i: (0, 0)),
            pl.BlockSpec((64, 32), lambda i: (0, 0)),
            pl.BlockSpec((1, 32), lambda i: (0, 0)),
        ],
        out_specs=pl.BlockSpec((_BB, 32), lambda i: (i, 0)),
        out_shape=jax.ShapeDtypeStruct((_B, 32), jnp.float32),
        interpret=interpret,
    )(emb, num, w1a, w1b, b1, w2, b2)


def kernel(cate_inputs, num_inputs, tables, W1, b1, W2, b2):
    v = cate_inputs.astype(jnp.int32)
    f_off = (jnp.arange(_F, dtype=jnp.int32) * _VP)[None, :]
    w = v % _VB
    idx = f_off + (v // _VB) * _VB + 4 * (w % _VQ) + w // _VQ
    tab_lin = _fmt(tables.transpose(0, 2, 1)).reshape(_F * _VP, _D)
    emb = _sc_gather_fn()(tab_lin, idx.reshape(-1))
    emb = emb.reshape(_B, _E)
    return _mlp(emb, num_inputs, W1[:_E], W1[_E:], b1.reshape(1, 64),
                W2, b2.reshape(1, 32))
